# initial kernel scaffold (unmeasured)
import functools

import jax
import jax.numpy as jnp
from jax import lax
from jax.experimental import pallas as pl
from jax.experimental.pallas import tpu as pltpu

N_DEV = 4


def kernel(x, w_mat):
    m, k = x.shape
    _, n = w_mat.shape
    m_blk = m // N_DEV

    def body(x_ref, w_ref, out_ref, comm_ref, send_sems, recv_sems):
        my = lax.axis_index("i")
        left = lax.rem(my + N_DEV - 1, N_DEV)
        right = lax.rem(my + 1, N_DEV)

        barrier = pltpu.get_barrier_semaphore()
        for nbr in (left, right):
            pl.semaphore_signal(barrier, inc=1, device_id=(nbr,),
                                device_id_type=pl.DeviceIdType.MESH)
        pl.semaphore_wait(barrier, 2)

        def block(b):
            xb = x_ref[pl.ds(b * m_blk, m_blk), :]
            return jnp.dot(xb, w_ref[:, :], preferred_element_type=jnp.float32)

        b0 = lax.rem(my + N_DEV - 1, N_DEV)
        comm_ref[0, :, :] = block(b0).astype(jnp.bfloat16)

        for h in range(N_DEV - 1):
            s, r = h, h + 1 if h + 1 < N_DEV - 1 else 0
            rdma = pltpu.make_async_remote_copy(
                src_ref=comm_ref.at[s],
                dst_ref=comm_ref.at[r],
                send_sem=send_sems.at[s],
                recv_sem=recv_sems.at[r],
                device_id=(right,),
                device_id_type=pl.DeviceIdType.MESH,
            )
            rdma.start()
            b = lax.rem(my + 2 * N_DEV - 2 - h, N_DEV)
            p = block(b)
            rdma.wait()
            if h < N_DEV - 2:
                comm_ref[r, :, :] = (
                    comm_ref[r, :, :].astype(jnp.float32) + p
                ).astype(jnp.bfloat16)
            else:
                out_ref[:, :] = jnp.maximum(
                    comm_ref[r, :, :].astype(jnp.float32) + p, 0.0
                )

        @functools.partial(
            pl.run_scoped, second_barrier=pltpu.SemaphoreType.REGULAR
        )
        def _(second_barrier):
            for nbr in (left, right):
                pl.semaphore_signal(second_barrier, inc=1, device_id=(nbr,),
                                    device_id_type=pl.DeviceIdType.MESH)
            pl.semaphore_wait(second_barrier, 2)

    return pl.pallas_call(
        body,
        out_shape=jax.ShapeDtypeStruct((m_blk, n), jnp.float32),
        in_specs=[
            pl.BlockSpec(memory_space=pltpu.VMEM),
            pl.BlockSpec(memory_space=pltpu.VMEM),
        ],
        out_specs=pl.BlockSpec(memory_space=pltpu.VMEM),
        scratch_shapes=[
            pltpu.VMEM((N_DEV - 1, m_blk, n), jnp.bfloat16),
            pltpu.SemaphoreType.DMA((N_DEV - 1,)),
            pltpu.SemaphoreType.DMA((N_DEV - 1,)),
        ],
        compiler_params=pltpu.CompilerParams(collective_id=0),
    )(x, w_mat)


# baseline (device time: 651529 ns/iter reference)
import functools

import jax

jax.config.update("jax_compilation_cache_dir", "/tmp/jax_kernel_cache")
jax.config.update("jax_persistent_cache_min_entry_size_bytes", -1)
jax.config.update("jax_persistent_cache_min_compile_time_secs", 0.0)

import jax.numpy as jnp
from jax import lax
from jax.experimental import pallas as pl
from jax.experimental.pallas import tpu as pltpu

N_DEV = 4
PIECES = 4
P_ROWS = 2048 // PIECES


def kernel(x, w_mat):
    x = x.astype(jnp.bfloat16)
    w = w_mat.astype(jnp.bfloat16)
    m, k_dim = x.shape
    n = w.shape[1]
    m_blk = m // N_DEV

    def body(x_ref, w_ref, out_ref, recv_bufs, xbuf, sbuf, rbuf, obuf,
             send_sems, recv_sems, xsems, rsems, osem):
        my = lax.axis_index("i")
        left = lax.rem(my + N_DEV - 1, N_DEV)
        right = lax.rem(my + 1, N_DEV)

        barrier = pltpu.get_barrier_semaphore()
        for nbr in (left, right):
            pl.semaphore_signal(barrier, inc=1, device_id=(nbr,),
                                device_id_type=pl.DeviceIdType.MESH)
        pl.semaphore_wait(barrier, 2)

        bs = [lax.rem(my + 2 * N_DEV - 1 - k, N_DEV) for k in range(N_DEV)]

        x_dmas = {}

        def start_x(i):
            if i >= N_DEV * PIECES:
                return
            k, s = divmod(i, PIECES)
            d = pltpu.make_async_copy(
                x_ref.at[pl.ds(bs[k] * m_blk + s * P_ROWS, P_ROWS), :],
                xbuf.at[i % 2],
                xsems.at[i % 2],
            )
            d.start()
            x_dmas[i] = d

        send_by_hop = {}
        send_by_n = {}
        out_dma = None

        for i in range(N_DEV * PIECES):
            k, s = divmod(i, PIECES)
            if i == 0:
                start_x(0)
            start_x(i + 1)
            x_dmas[i].wait()
            if k > 0:
                send_by_hop[(k - 1, s)].wait_recv()
                rd = pltpu.make_async_copy(
                    recv_bufs.at[k - 1, s], rbuf.at[s % 2], rsems.at[s % 2]
                )
                rd.start()
            p = jnp.dot(xbuf[i % 2], w_ref[...],
                        preferred_element_type=jnp.float32)
            if k > 0:
                rd.wait()
                p = p + rbuf[s % 2].astype(jnp.float32)
            if k < N_DEV - 1:
                nsend = k * PIECES + s
                if nsend >= 2:
                    send_by_n[nsend - 2].wait_send()
                sbuf[nsend % 2] = p.astype(jnp.bfloat16)
                d = pltpu.make_async_remote_copy(
                    src_ref=sbuf.at[nsend % 2],
                    dst_ref=recv_bufs.at[k, s],
                    send_sem=send_sems.at[k, s],
                    recv_sem=recv_sems.at[k, s],
                    device_id=(right,),
                    device_id_type=pl.DeviceIdType.MESH,
                )
                d.start()
                send_by_hop[(k, s)] = d
                send_by_n[nsend] = d
            else:
                if out_dma is not None:
                    out_dma.wait()
                obuf[...] = jnp.maximum(p, 0.0)
                out_dma = pltpu.make_async_copy(
                    obuf, out_ref.at[pl.ds(s * P_ROWS, P_ROWS), :], osem
                )
                out_dma.start()
        out_dma.wait()
        n_sends = (N_DEV - 1) * PIECES
        send_by_n[n_sends - 2].wait_send()
        send_by_n[n_sends - 1].wait_send()

        @functools.partial(
            pl.run_scoped, second_barrier=pltpu.SemaphoreType.REGULAR
        )
        def _(second_barrier):
            for nbr in (left, right):
                pl.semaphore_signal(second_barrier, inc=1, device_id=(nbr,),
                                    device_id_type=pl.DeviceIdType.MESH)
            pl.semaphore_wait(second_barrier, 2)

    hbm = pltpu.MemorySpace.HBM
    vmem = pltpu.MemorySpace.VMEM
    out, _ = pl.pallas_call(
        body,
        out_shape=[
            jax.ShapeDtypeStruct((m_blk, n), jnp.float32),
            jax.ShapeDtypeStruct((N_DEV - 1, PIECES, P_ROWS, n),
                                 jnp.bfloat16),
        ],
        in_specs=[
            pl.BlockSpec(memory_space=hbm),
            pl.BlockSpec(memory_space=vmem),
        ],
        out_specs=[
            pl.BlockSpec(memory_space=hbm),
            pl.BlockSpec(memory_space=hbm),
        ],
        scratch_shapes=[
            vmem((2, P_ROWS, k_dim), jnp.bfloat16),
            vmem((2, P_ROWS, n), jnp.bfloat16),
            vmem((2, P_ROWS, n), jnp.bfloat16),
            vmem((P_ROWS, n), jnp.float32),
            pltpu.SemaphoreType.DMA((N_DEV - 1, PIECES)),
            pltpu.SemaphoreType.DMA((N_DEV - 1, PIECES)),
            pltpu.SemaphoreType.DMA((2,)),
            pltpu.SemaphoreType.DMA((2,)),
            pltpu.SemaphoreType.DMA,
        ],
        compiler_params=pltpu.CompilerParams(
            collective_id=0, vmem_limit_bytes=100 * 1024 * 1024
        ),
    )(x, w)
    return out
